# Initial kernel scaffold; baseline (speedup 1.0000x reference)
import functools

import jax
import jax.numpy as jnp
from jax import lax
from jax.experimental import pallas as pl
from jax.experimental.pallas import tpu as pltpu
from jax.experimental.pallas import tpu_sc as plsc

_B = 16384 * 200
_D = 32
_NC = 2
_NS = 16
_NW = _NC * _NS
_BPW = _B // _NW
_CHUNK = 1024
_NIT = _BPW // _CHUNK


def _make_gather():
    mesh = plsc.VectorSubcoreMesh(core_axis_name="c", subcore_axis_name="s")

    @functools.partial(
        pl.kernel,
        mesh=mesh,
        out_type=jax.ShapeDtypeStruct((_B, _D), jnp.float32),
        scratch_types=[
            pltpu.VMEM((_CHUNK,), jnp.int32),
            pltpu.VMEM((_CHUNK, _D), jnp.float32),
            pltpu.SemaphoreType.DMA,
        ],
    )
    def gather(idx_hbm, table_hbm, out_hbm, idx_v, rows_v, sem):
        wid = lax.axis_index("s") * _NC + lax.axis_index("c")
        base = wid * _BPW

        def step(it, carry):
            off = pl.multiple_of(base + it * _CHUNK, _CHUNK)
            pltpu.sync_copy(idx_hbm.at[pl.ds(off, _CHUNK)], idx_v)
            pltpu.async_copy(table_hbm.at[idx_v], rows_v, sem).wait()
            pltpu.sync_copy(rows_v, out_hbm.at[pl.ds(off, _CHUNK)])
            return carry

        lax.fori_loop(0, _NIT, step, 0)

    return gather


_gather = _make_gather()


def kernel(x, weight):
    idx = x.reshape(-1).astype(jnp.int32)
    out = _gather(idx, weight)
    return out.reshape(x.shape + (weight.shape[1],))


# speedup vs baseline: 4.8091x; 4.8091x over previous
import functools

import jax
import jax.numpy as jnp
from jax import lax
from jax.experimental import pallas as pl
from jax.experimental.pallas import tpu as pltpu
from jax.experimental.pallas import tpu_sc as plsc

_B = 16384 * 200
_D = 32
_NC = 2
_NS = 16
_NW = _NC * _NS
_BPW = _B // _NW
_CHUNK = 1024
_NIT = _BPW // _CHUNK


def _make_gather():
    mesh = plsc.VectorSubcoreMesh(core_axis_name="c", subcore_axis_name="s")

    @functools.partial(
        pl.kernel,
        mesh=mesh,
        out_type=jax.ShapeDtypeStruct((_B, _D), jnp.float32),
        scratch_types=[
            pltpu.VMEM((_CHUNK,), jnp.int32),
            pltpu.VMEM((_CHUNK, _D), jnp.float32),
            pltpu.SemaphoreType.DMA,
        ],
        compiler_params=pltpu.CompilerParams(use_tc_tiling_on_sc=False),
    )
    def gather(idx_hbm, table_hbm, out_hbm, idx_v, rows_v, sem):
        wid = lax.axis_index("s") * _NC + lax.axis_index("c")
        base = wid * _BPW

        def step(it, carry):
            off = pl.multiple_of(base + it * _CHUNK, _CHUNK)
            pltpu.sync_copy(idx_hbm.at[pl.ds(off, _CHUNK)], idx_v)
            pltpu.async_copy(table_hbm.at[idx_v], rows_v, sem).wait()
            pltpu.sync_copy(rows_v, out_hbm.at[pl.ds(off, _CHUNK)])
            return carry

        lax.fori_loop(0, _NIT, step, 0)

    return gather


_gather = _make_gather()


def kernel(x, weight):
    idx = x.reshape(-1).astype(jnp.int32)
    out = _gather(idx, weight)
    return out.reshape(x.shape + (weight.shape[1],))


# speedup vs baseline: 5.0093x; 1.0416x over previous
import functools

import jax
import jax.numpy as jnp
from jax import lax
from jax.experimental import pallas as pl
from jax.experimental.pallas import tpu as pltpu
from jax.experimental.pallas import tpu_sc as plsc

_B = 16384 * 200
_D = 32
_NC = 2
_NS = 16
_NW = _NC * _NS
_BPW = _B // _NW
_CHUNK = 800
_NIT = _BPW // _CHUNK
_NBUF = 4
_NG = _NIT // _NBUF

assert _BPW % _CHUNK == 0 and _NIT % _NBUF == 0 and _CHUNK % 8 == 0


def _make_gather():
    mesh = plsc.VectorSubcoreMesh(core_axis_name="c", subcore_axis_name="s")

    scratch = (
        [pltpu.VMEM((_CHUNK,), jnp.int32) for _ in range(_NBUF)]
        + [pltpu.VMEM((_CHUNK, _D), jnp.float32) for _ in range(_NBUF)]
        + [pltpu.SemaphoreType.DMA for _ in range(_NBUF)]
        + [pltpu.SemaphoreType.DMA for _ in range(_NBUF)]
        + [pltpu.SemaphoreType.DMA]
    )

    @functools.partial(
        pl.kernel,
        mesh=mesh,
        out_type=jax.ShapeDtypeStruct((_B, _D), jnp.float32),
        scratch_types=scratch,
        compiler_params=pltpu.CompilerParams(use_tc_tiling_on_sc=False),
    )
    def gather(idx_hbm, table_hbm, out_hbm, *refs):
        idx_bufs = refs[0:_NBUF]
        row_bufs = refs[_NBUF:2 * _NBUF]
        isems = refs[2 * _NBUF:3 * _NBUF]
        osems = refs[3 * _NBUF:4 * _NBUF]
        gsem = refs[4 * _NBUF]

        wid = lax.axis_index("s") * _NC + lax.axis_index("c")
        base = wid * _BPW

        def off(it):
            return pl.multiple_of(base + it * _CHUNK, 32)

        def wait_osem(b):
            pltpu.make_async_copy(
                row_bufs[b], out_hbm.at[pl.ds(0, _CHUNK)], osems[b]).wait()

        def wait_isem(b):
            pltpu.make_async_copy(
                idx_hbm.at[pl.ds(0, _CHUNK)], idx_bufs[b], isems[b]).wait()

        def body(it, b, first, last):
            bn = (b + 1) % _NBUF
            if not first:
                wait_osem(b)
            if not last:
                pltpu.async_copy(
                    idx_hbm.at[pl.ds(off(it + 1), _CHUNK)], idx_bufs[bn],
                    isems[bn])
            wait_isem(b)
            pltpu.async_copy(
                table_hbm.at[idx_bufs[b]], row_bufs[b], gsem).wait()
            pltpu.async_copy(
                row_bufs[b], out_hbm.at[pl.ds(off(it), _CHUNK)], osems[b])

        pltpu.async_copy(idx_hbm.at[pl.ds(off(0), _CHUNK)], idx_bufs[0],
                         isems[0])

        for b in range(_NBUF):
            body(b, b, first=True, last=False)

        def group(g, carry):
            for b in range(_NBUF):
                body(g * _NBUF + b, b, first=False, last=False)
            return carry

        lax.fori_loop(1, _NG - 1, group, 0)

        for b in range(_NBUF):
            it = (_NG - 1) * _NBUF + b
            body(it, b, first=False, last=(b == _NBUF - 1))

        for b in range(_NBUF):
            wait_osem(b)

    return gather


_gather = _make_gather()


def kernel(x, weight):
    idx = x.reshape(-1).astype(jnp.int32)
    out = _gather(idx, weight)
    return out.reshape(x.shape + (weight.shape[1],))
